# Initial kernel scaffold; baseline (speedup 1.0000x reference)
import jax
import jax.numpy as jnp
from jax.experimental import pallas as pl


def kernel(anum, edge_index, edge_to_src, dist, table, Wg, Wi, bi, Wo, bo):
    raise NotImplementedError("write your pallas kernel here")


# speedup vs baseline: 6.9110x; 6.9110x over previous
import functools

import jax
import jax.numpy as jnp
from jax import lax
from jax.experimental import pallas as pl
from jax.experimental.pallas import tpu as pltpu
from jax.experimental.pallas import tpu_sc as plsc

N = 10000
E = 160000
NUM_ELEM = 100
EMBED = 256
HID = 512
NG = 50
NGP = 64
NH = 8
NM = 1
RBF_R = 12.0

NC = 2
NS = 16
NW = NC * NS
L = 16
EPW = E // NW
PAD = 8
CH = 200
NCHUNK = EPW // CH

TE = 1280
GRID = E // TE

_STEP = RBF_R / (NG - 1)
_COEFF = -0.5 / _STEP ** 2


def _sc_gather_body(anum_hbm, e0_hbm, e1_hbm, src_hbm, table_hbm, out_hbm,
                    anum_v, idx_v, s0_v, s1_v, rows_v, sem):
    wid = lax.axis_index("s") * NC + lax.axis_index("c")
    base = wid * EPW
    pltpu.sync_copy(anum_hbm, anum_v)
    idx_v[pl.ds(EPW + PAD - L, L)] = jnp.zeros((L,), jnp.int32)
    pltpu.sync_copy(src_hbm.at[pl.ds(base, EPW)], idx_v.at[pl.ds(0, EPW)])
    pltpu.async_copy(e0_hbm.at[idx_v], s0_v, sem).wait()
    pltpu.async_copy(e1_hbm.at[idx_v], s1_v, sem).wait()

    def pair_step(i, carry):
        sl = pl.ds(i * L, L)
        s0 = jnp.clip(s0_v[sl], 0, N - 1)
        s1 = jnp.clip(s1_v[sl], 0, N - 1)
        a0 = plsc.load_gather(anum_v, [s0])
        a1 = plsc.load_gather(anum_v, [s1])
        idx_v[sl] = a0 + NUM_ELEM * a1
        return carry

    lax.fori_loop(0, (EPW + PAD) // L, pair_step, 0)

    def chunk_step(c, carry):
        pltpu.async_copy(
            table_hbm.at[idx_v.at[pl.ds(c * CH, CH)]], rows_v, sem).wait()
        pltpu.sync_copy(rows_v, out_hbm.at[pl.ds(base + c * CH, CH)])
        return carry

    lax.fori_loop(0, NCHUNK, chunk_step, 0)


_sc_gather = functools.partial(
    pl.kernel,
    out_type=jax.ShapeDtypeStruct((E, EMBED), jnp.float32),
    mesh=plsc.VectorSubcoreMesh(core_axis_name="c", subcore_axis_name="s"),
    compiler_params=pltpu.CompilerParams(needs_layout_passes=False),
    scratch_types=[
        pltpu.VMEM((N,), jnp.int32),
        pltpu.VMEM((EPW + PAD,), jnp.int32),
        pltpu.VMEM((EPW + PAD,), jnp.int32),
        pltpu.VMEM((EPW + PAD,), jnp.int32),
        pltpu.VMEM((CH, EMBED), jnp.float32),
        pltpu.SemaphoreType.DMA,
    ],
)(_sc_gather_body)


def _tc_body(dist_ref, emb_ref, wg_ref, wie_ref, wir_ref, bi_ref, wo_ref,
             bo_ref, out_ref):
    d = dist_ref[...]
    col = lax.broadcasted_iota(jnp.int32, (TE, NGP), 1).astype(jnp.float32) * _STEP
    diff = d - col
    rbf = jnp.exp(_COEFF * diff * diff)
    dn = (((1,), (1,)), ((), ()))
    gate = lax.dot_general(rbf, wg_ref[...], dn,
                           preferred_element_type=jnp.float32)
    acc = lax.dot_general(emb_ref[...], wie_ref[...], dn,
                          preferred_element_type=jnp.float32)
    acc = acc + lax.dot_general(rbf, wir_ref[...], dn,
                                preferred_element_type=jnp.float32)
    acc = acc + bi_ref[...]
    h = acc * (1.0 / (1.0 + jnp.exp(-acc))) * gate
    o = lax.dot_general(wo_ref[...], h, dn,
                        preferred_element_type=jnp.float32)
    out_ref[...] = o + bo_ref[...]


_tc_mlp = pl.pallas_call(
    _tc_body,
    grid=(GRID,),
    in_specs=[
        pl.BlockSpec((TE, 1), lambda i: (i, 0)),
        pl.BlockSpec((TE, EMBED), lambda i: (i, 0)),
        pl.BlockSpec((HID, NGP), lambda i: (0, 0)),
        pl.BlockSpec((HID, EMBED), lambda i: (0, 0)),
        pl.BlockSpec((HID, NGP), lambda i: (0, 0)),
        pl.BlockSpec((1, HID), lambda i: (0, 0)),
        pl.BlockSpec((NH, HID), lambda i: (0, 0)),
        pl.BlockSpec((NH, 1), lambda i: (0, 0)),
    ],
    out_specs=pl.BlockSpec((NH, TE), lambda i: (0, i)),
    out_shape=jax.ShapeDtypeStruct((NH, E), jnp.float32),
)


def kernel(anum, edge_index, edge_to_src, dist, table, Wg, Wi, bi, Wo, bo):
    anum = anum.astype(jnp.int32)
    e0 = edge_index[0].astype(jnp.int32)
    e1 = edge_index[1].astype(jnp.int32)
    src = edge_to_src.astype(jnp.int32)
    emb = _sc_gather(anum, e0, e1, src, table)
    wg_p = jnp.pad(Wg, ((0, 0), (0, NGP - NG)))
    wi_emb = Wi[:, :EMBED]
    wi_rbf = jnp.pad(Wi[:, EMBED:], ((0, 0), (0, NGP - NG)))
    out2d = _tc_mlp(dist.reshape(E, 1), emb, wg_p, wi_emb, wi_rbf,
                    bi.reshape(1, HID), Wo, bo.reshape(NH, 1))
    return out2d.reshape(NM, NH, E)
